# baseline (speedup 1.0000x reference)
import functools

import jax
import jax.numpy as jnp
from jax import lax
from jax.experimental import pallas as pl
from jax.experimental.pallas import tpu as pltpu
from jax.experimental.pallas import tpu_sc as plsc

EMB = 128
EDGES = 320000
NC, NS = 2, 16
NW = NC * NS
PER_W = EDGES // NW
CHUNK = 200
NCH = PER_W // CHUNK
SUBS = ((0, 128), (128, 72))
TROWS = 776
EBLK = 2048


def _tc_prep(ea, W0, W1, W2):
    def body(ea_blk, w0, w1, w2, t, idx):
        @pl.when(pl.program_id(0) == 0)
        def _build_table():
            def onehot(n, row_of_i):
                i = lax.broadcasted_iota(jnp.int32, (TROWS, n), 0)
                j = lax.broadcasted_iota(jnp.int32, (TROWS, n), 1)
                return (row_of_i(i) == j).astype(jnp.float32)

            a0 = onehot(10, lambda i: i // 77)
            a1 = onehot(11, lambda i: (i // 7) % 11)
            a2 = onehot(7, lambda i: i % 7)
            t[...] = (
                jnp.dot(a0, w0[...], preferred_element_type=jnp.float32,
                        precision=lax.Precision.HIGHEST)
                + jnp.dot(a1, w1[...], preferred_element_type=jnp.float32,
                          precision=lax.Precision.HIGHEST)
                + jnp.dot(a2, w2[...], preferred_element_type=jnp.float32,
                          precision=lax.Precision.HIGHEST)
            )

        e = ea_blk[...]
        idx[...] = e[:, 0] * 77 + e[:, 1] * 7 + e[:, 2]

    return pl.pallas_call(
        body,
        grid=(pl.cdiv(EDGES, EBLK),),
        in_specs=[
            pl.BlockSpec((EBLK, 3), lambda i: (i, 0)),
            pl.BlockSpec((10, EMB), lambda i: (0, 0)),
            pl.BlockSpec((11, EMB), lambda i: (0, 0)),
            pl.BlockSpec((7, EMB), lambda i: (0, 0)),
        ],
        out_specs=[
            pl.BlockSpec((TROWS, EMB), lambda i: (0, 0)),
            pl.BlockSpec((EBLK,), lambda i: (i,)),
        ],
        out_shape=[
            jax.ShapeDtypeStruct((TROWS, EMB), jnp.float32),
            jax.ShapeDtypeStruct((EDGES,), jnp.int32),
        ],
    )(ea, W0, W1, W2)


_MESH = plsc.VectorSubcoreMesh(core_axis_name="c", subcore_axis_name="s")


@functools.partial(
    pl.kernel,
    mesh=_MESH,
    out_type=jax.ShapeDtypeStruct((EDGES, EMB), jnp.float32),
    scratch_types=[
        pltpu.VMEM((PER_W,), jnp.int32),
        pltpu.VMEM((CHUNK, EMB), jnp.float32),
        pltpu.VMEM((CHUNK, EMB), jnp.float32),
        pltpu.SemaphoreType.DMA,
        pltpu.SemaphoreType.DMA,
        pltpu.SemaphoreType.DMA,
        pltpu.SemaphoreType.DMA,
        pltpu.VMEM_SHARED((TROWS, EMB), jnp.float32),
    ],
)
def _sc_lookup(idx_hbm, t_hbm, out_hbm,
               idx_v, rows0_v, rows1_v,
               gsem0, gsem1, wsem0, wsem1, t_sh):
    sid = lax.axis_index("s")
    wid = sid * NC + lax.axis_index("c")
    base = wid * PER_W

    @pl.when(sid == 0)
    def _stage_table():
        pltpu.sync_copy(t_hbm, t_sh)

    pltpu.sync_copy(idx_hbm.at[pl.ds(base, PER_W)], idx_v)
    plsc.subcore_barrier()

    def issue_gathers(c, buf, sem):
        r0 = c * CHUNK
        return [
            pltpu.async_copy(
                t_sh.at[idx_v.at[pl.ds(r0 + off, n)]],
                buf.at[pl.ds(off, n), :],
                sem,
            )
            for off, n in SUBS
        ]

    def issue_write(c, buf, sem):
        pltpu.async_copy(buf, out_hbm.at[pl.ds(base + c * CHUNK, CHUNK), :], sem)

    def wait_write(buf, sem):
        pltpu.make_async_copy(buf, out_hbm.at[pl.ds(base, CHUNK), :], sem).wait()

    def do_pair(k, first):
        c = 2 * k
        if not first:
            wait_write(rows0_v, wsem0)
        g0 = issue_gathers(c, rows0_v, gsem0)
        if not first:
            wait_write(rows1_v, wsem1)
        g1 = issue_gathers(c + 1, rows1_v, gsem1)
        for cp in g0:
            cp.wait()
        issue_write(c, rows0_v, wsem0)
        for cp in g1:
            cp.wait()
        issue_write(c + 1, rows1_v, wsem1)

    do_pair(0, first=True)

    def pair_body(k, carry):
        do_pair(k, first=False)
        return carry

    lax.fori_loop(1, NCH // 2, pair_body, 0)
    wait_write(rows0_v, wsem0)
    wait_write(rows1_v, wsem1)


def kernel(edge_attr, W0, W1, W2):
    ea = edge_attr.astype(jnp.int32)
    t, idx = _tc_prep(ea, W0, W1, W2)
    return _sc_lookup(idx, t)


# speedup vs baseline: 1.3907x; 1.3907x over previous
import functools

import jax
import jax.numpy as jnp
from jax import lax
from jax.experimental import pallas as pl
from jax.experimental.pallas import tpu as pltpu
from jax.experimental.pallas import tpu_sc as plsc

EMB = 128
EDGES = 320000
NC, NS = 2, 16
NW = NC * NS
PER_W = EDGES // NW
CHUNK = 200
NCH = PER_W // CHUNK
SUBS = ((0, 128), (128, 72))
TROWS = 776
EROWS = EDGES // EMB


def _tc_prep(ea, W0, W1, W2):
    def body(ea_blk, w0, w1, w2, t, idx):
        def _build_table():
            def onehot(n, row_of_i):
                i = lax.broadcasted_iota(jnp.int32, (TROWS, n), 0)
                j = lax.broadcasted_iota(jnp.int32, (TROWS, n), 1)
                return (row_of_i(i) == j).astype(jnp.float32)

            a0 = onehot(10, lambda i: i // 77)
            a1 = onehot(11, lambda i: (i // 7) % 11)
            a2 = onehot(7, lambda i: i % 7)
            t[...] = (
                jnp.dot(a0, w0[...], preferred_element_type=jnp.float32,
                        precision=lax.Precision.HIGHEST)
                + jnp.dot(a1, w1[...], preferred_element_type=jnp.float32,
                          precision=lax.Precision.HIGHEST)
                + jnp.dot(a2, w2[...], preferred_element_type=jnp.float32,
                          precision=lax.Precision.HIGHEST)
            )

        _build_table()
        jj = lax.broadcasted_iota(jnp.int32, (3 * EMB, EMB), 0)
        ll = lax.broadcasted_iota(jnp.int32, (3 * EMB, EMB), 1)
        c = jj % 3
        w = jnp.where(c == 0, 77, jnp.where(c == 1, 7, 1))
        m = jnp.where(jj // 3 == ll, w, 0).astype(jnp.float32)
        e = ea_blk[...].astype(jnp.float32)
        idx[...] = jnp.dot(
            e, m, preferred_element_type=jnp.float32,
            precision=lax.Precision.HIGHEST,
        ).astype(jnp.int32)

    return pl.pallas_call(
        body,
        out_shape=[
            jax.ShapeDtypeStruct((TROWS, EMB), jnp.float32),
            jax.ShapeDtypeStruct((EROWS, EMB), jnp.int32),
        ],
    )(ea, W0, W1, W2)


_MESH = plsc.VectorSubcoreMesh(core_axis_name="c", subcore_axis_name="s")


@functools.partial(
    pl.kernel,
    mesh=_MESH,
    out_type=jax.ShapeDtypeStruct((EDGES, EMB), jnp.float32),
    scratch_types=[
        pltpu.VMEM((PER_W,), jnp.int32),
        pltpu.VMEM((CHUNK, EMB), jnp.float32),
        pltpu.VMEM((CHUNK, EMB), jnp.float32),
        pltpu.SemaphoreType.DMA,
        pltpu.SemaphoreType.DMA,
        pltpu.SemaphoreType.DMA,
        pltpu.SemaphoreType.DMA,
        pltpu.VMEM_SHARED((TROWS, EMB), jnp.float32),
    ],
)
def _sc_lookup(idx_hbm, t_hbm, out_hbm,
               idx_v, rows0_v, rows1_v,
               gsem0, gsem1, wsem0, wsem1, t_sh):
    sid = lax.axis_index("s")
    wid = sid * NC + lax.axis_index("c")
    base = wid * PER_W

    @pl.when(sid == 0)
    def _stage_table():
        pltpu.sync_copy(t_hbm, t_sh)

    pltpu.sync_copy(idx_hbm.at[pl.ds(base, PER_W)], idx_v)
    plsc.subcore_barrier()

    def issue_gathers(c, buf, sem):
        r0 = c * CHUNK
        return [
            pltpu.async_copy(
                t_sh.at[idx_v.at[pl.ds(r0 + off, n)]],
                buf.at[pl.ds(off, n), :],
                sem,
            )
            for off, n in SUBS
        ]

    def issue_write(c, buf, sem):
        pltpu.async_copy(buf, out_hbm.at[pl.ds(base + c * CHUNK, CHUNK), :], sem)

    def wait_write(buf, sem):
        pltpu.make_async_copy(buf, out_hbm.at[pl.ds(base, CHUNK), :], sem).wait()

    def do_pair(k, first):
        c = 2 * k
        if not first:
            wait_write(rows0_v, wsem0)
        g0 = issue_gathers(c, rows0_v, gsem0)
        if not first:
            wait_write(rows1_v, wsem1)
        g1 = issue_gathers(c + 1, rows1_v, gsem1)
        for cp in g0:
            cp.wait()
        issue_write(c, rows0_v, wsem0)
        for cp in g1:
            cp.wait()
        issue_write(c + 1, rows1_v, wsem1)

    do_pair(0, first=True)

    def pair_body(k, carry):
        do_pair(k, first=False)
        return carry

    lax.fori_loop(1, NCH // 2, pair_body, 0)
    wait_write(rows0_v, wsem0)
    wait_write(rows1_v, wsem1)


def kernel(edge_attr, W0, W1, W2):
    ea = edge_attr.astype(jnp.int32).reshape(EROWS, 3 * EMB)
    t, idx = _tc_prep(ea, W0, W1, W2)
    return _sc_lookup(idx.reshape(-1), t)


# speedup vs baseline: 2.9645x; 2.1316x over previous
import functools

import jax
import jax.numpy as jnp
from jax import lax
from jax.experimental import pallas as pl
from jax.experimental.pallas import tpu as pltpu
from jax.experimental.pallas import tpu_sc as plsc

EMB = 128
EDGES = 320000
NC, NS = 2, 16
NW = NC * NS
PER_W = EDGES // NW
CHUNK = 200
NCH = PER_W // CHUNK
SUBS = ((0, 128), (128, 72))
STAGE = 2000
NST = PER_W // STAGE
TROWS = 776


def _build_table(W0, W1, W2):
    def body(w0, w1, w2, t):
        def onehot(n, row_of_i):
            i = lax.broadcasted_iota(jnp.int32, (TROWS, n), 0)
            j = lax.broadcasted_iota(jnp.int32, (TROWS, n), 1)
            return (row_of_i(i) == j).astype(jnp.float32)

        a0 = onehot(10, lambda i: i // 77)
        a1 = onehot(11, lambda i: (i // 7) % 11)
        a2 = onehot(7, lambda i: i % 7)
        t[...] = (
            jnp.dot(a0, w0[...], preferred_element_type=jnp.float32, precision=lax.Precision.HIGHEST)
            + jnp.dot(a1, w1[...], preferred_element_type=jnp.float32, precision=lax.Precision.HIGHEST)
            + jnp.dot(a2, w2[...], preferred_element_type=jnp.float32, precision=lax.Precision.HIGHEST)
        )

    return pl.pallas_call(
        body,
        out_shape=jax.ShapeDtypeStruct((TROWS, EMB), jnp.float32),
    )(W0, W1, W2)


_MESH = plsc.VectorSubcoreMesh(core_axis_name="c", subcore_axis_name="s")


@functools.partial(
    pl.kernel,
    mesh=_MESH,
    out_type=jax.ShapeDtypeStruct((EDGES, EMB), jnp.float32),
    scratch_types=[
        pltpu.VMEM((PER_W,), jnp.int32),
        pltpu.VMEM((PER_W,), jnp.int32),
        pltpu.VMEM((PER_W,), jnp.int32),
        pltpu.VMEM((PER_W,), jnp.int32),
        pltpu.VMEM((CHUNK, EMB), jnp.float32),
        pltpu.VMEM((CHUNK, EMB), jnp.float32),
        pltpu.SemaphoreType.DMA,
        pltpu.SemaphoreType.DMA,
        pltpu.SemaphoreType.DMA,
        pltpu.SemaphoreType.DMA,
        pltpu.VMEM_SHARED((TROWS, EMB), jnp.float32),
    ],
)
def _sc_lookup(a0_hbm, a1_hbm, a2_hbm, t_hbm, out_hbm,
               a0_v, a1_v, a2_v, idx_v, rows0_v, rows1_v,
               gsem0, gsem1, wsem0, wsem1, t_sh):
    sid = lax.axis_index("s")
    wid = sid * NC + lax.axis_index("c")
    base = wid * PER_W

    @pl.when(sid == 0)
    def _stage_table():
        pltpu.sync_copy(t_hbm, t_sh)

    cols = [
        pltpu.async_copy(src.at[pl.ds(base, PER_W)], dst, gsem0)
        for src, dst in ((a0_hbm, a0_v), (a1_hbm, a1_v), (a2_hbm, a2_v))
    ]
    for cp in cols:
        cp.wait()

    def vec_body(i, carry):
        s = pl.ds(i * 16, 16)
        idx_v[s] = a0_v[s] * 77 + a1_v[s] * 7 + a2_v[s]
        return carry

    lax.fori_loop(0, PER_W // 16, vec_body, 0)
    plsc.subcore_barrier()

    def issue_gathers(c, buf, sem):
        r0 = c * CHUNK
        return [
            pltpu.async_copy(
                t_sh.at[idx_v.at[pl.ds(r0 + off, n)]],
                buf.at[pl.ds(off, n), :],
                sem,
            )
            for off, n in SUBS
        ]

    def issue_write(c, buf, sem):
        pltpu.async_copy(buf, out_hbm.at[pl.ds(base + c * CHUNK, CHUNK), :], sem)

    def wait_write(buf, sem):
        pltpu.make_async_copy(buf, out_hbm.at[pl.ds(base, CHUNK), :], sem).wait()

    def do_pair(k, first):
        c = 2 * k
        if not first:
            wait_write(rows0_v, wsem0)
        g0 = issue_gathers(c, rows0_v, gsem0)
        if not first:
            wait_write(rows1_v, wsem1)
        g1 = issue_gathers(c + 1, rows1_v, gsem1)
        for cp in g0:
            cp.wait()
        issue_write(c, rows0_v, wsem0)
        for cp in g1:
            cp.wait()
        issue_write(c + 1, rows1_v, wsem1)

    do_pair(0, first=True)

    def pair_body(k, carry):
        do_pair(k, first=False)
        return carry

    lax.fori_loop(1, NCH // 2, pair_body, 0)
    wait_write(rows0_v, wsem0)
    wait_write(rows1_v, wsem1)


def kernel(edge_attr, W0, W1, W2):
    ea = edge_attr.astype(jnp.int32)
    t = _build_table(W0, W1, W2)
    return _sc_lookup(ea[:, 0], ea[:, 1], ea[:, 2], t)


# speedup vs baseline: 2.9990x; 1.0116x over previous
import functools

import jax
import jax.numpy as jnp
from jax import lax
from jax.experimental import pallas as pl
from jax.experimental.pallas import tpu as pltpu
from jax.experimental.pallas import tpu_sc as plsc

EMB = 128
EDGES = 320000
NC, NS = 2, 16
NW = NC * NS
PER_W = EDGES // NW
CHUNK = 200
NCH = PER_W // CHUNK
SUBS = ((0, 128), (128, 72))
STAGE = 2000
NST = PER_W // STAGE
TROWS = 776


def _build_table(W0, W1, W2):
    def body(w0, w1, w2, t):
        def onehot(n, row_of_i):
            i = lax.broadcasted_iota(jnp.int32, (TROWS, n), 0)
            j = lax.broadcasted_iota(jnp.int32, (TROWS, n), 1)
            return (row_of_i(i) == j).astype(jnp.float32)

        a0 = onehot(10, lambda i: i // 77)
        a1 = onehot(11, lambda i: (i // 7) % 11)
        a2 = onehot(7, lambda i: i % 7)
        t[...] = (
            jnp.dot(a0, w0[...], preferred_element_type=jnp.float32, precision=lax.Precision.HIGHEST)
            + jnp.dot(a1, w1[...], preferred_element_type=jnp.float32, precision=lax.Precision.HIGHEST)
            + jnp.dot(a2, w2[...], preferred_element_type=jnp.float32, precision=lax.Precision.HIGHEST)
        )

    return pl.pallas_call(
        body,
        out_shape=jax.ShapeDtypeStruct((TROWS, EMB), jnp.float32),
    )(W0, W1, W2)


_MESH = plsc.VectorSubcoreMesh(core_axis_name="c", subcore_axis_name="s")


@functools.partial(
    pl.kernel,
    mesh=_MESH,
    out_type=jax.ShapeDtypeStruct((EDGES, EMB), jnp.float32),
    scratch_types=[
        pltpu.VMEM((PER_W,), jnp.int32),
        pltpu.VMEM((PER_W,), jnp.int32),
        pltpu.VMEM((PER_W,), jnp.int32),
        pltpu.VMEM((PER_W,), jnp.int32),
        pltpu.VMEM((CHUNK, EMB), jnp.float32),
        pltpu.VMEM((CHUNK, EMB), jnp.float32),
        pltpu.SemaphoreType.DMA,
        pltpu.SemaphoreType.DMA,
        pltpu.SemaphoreType.DMA,
        pltpu.SemaphoreType.DMA,
        pltpu.VMEM_SHARED((TROWS, EMB), jnp.float32),
    ],
)
def _sc_lookup(a0_hbm, a1_hbm, a2_hbm, t_hbm, out_hbm,
               a0_v, a1_v, a2_v, idx_v, rows0_v, rows1_v,
               gsem0, gsem1, wsem0, wsem1, t_sh):
    sid = lax.axis_index("s")
    wid = sid * NC + lax.axis_index("c")
    base = wid * PER_W

    @pl.when(sid == 0)
    def _stage_table():
        pltpu.sync_copy(t_hbm, t_sh)

    cols = [
        pltpu.async_copy(src.at[pl.ds(base, PER_W)], dst, gsem0)
        for src, dst in ((a0_hbm, a0_v), (a1_hbm, a1_v), (a2_hbm, a2_v))
    ]
    for cp in cols:
        cp.wait()

    def vec_body(i, carry):
        s = pl.ds(i * 16, 16)
        idx_v[s] = a0_v[s] * 77 + a1_v[s] * 7 + a2_v[s]
        return carry

    HEAD = 2 * CHUNK // 16
    lax.fori_loop(0, HEAD, vec_body, 0)
    plsc.subcore_barrier()

    def issue_gathers(c, buf, sem):
        r0 = c * CHUNK
        return [
            pltpu.async_copy(
                t_sh.at[idx_v.at[pl.ds(r0 + off, n)]],
                buf.at[pl.ds(off, n), :],
                sem,
            )
            for off, n in SUBS
        ]

    def issue_write(c, buf, sem):
        pltpu.async_copy(buf, out_hbm.at[pl.ds(base + c * CHUNK, CHUNK), :], sem)

    def wait_write(buf, sem):
        pltpu.make_async_copy(buf, out_hbm.at[pl.ds(base, CHUNK), :], sem).wait()

    def do_pair(k, first):
        c = 2 * k
        if not first:
            wait_write(rows0_v, wsem0)
        g0 = issue_gathers(c, rows0_v, gsem0)
        if not first:
            wait_write(rows1_v, wsem1)
        g1 = issue_gathers(c + 1, rows1_v, gsem1)
        for cp in g0:
            cp.wait()
        issue_write(c, rows0_v, wsem0)
        for cp in g1:
            cp.wait()
        issue_write(c + 1, rows1_v, wsem1)

    g0 = issue_gathers(0, rows0_v, gsem0)
    g1 = issue_gathers(1, rows1_v, gsem1)
    lax.fori_loop(HEAD, PER_W // 16, vec_body, 0)
    for cp in g0:
        cp.wait()
    issue_write(0, rows0_v, wsem0)
    for cp in g1:
        cp.wait()
    issue_write(1, rows1_v, wsem1)

    def pair_body(k, carry):
        do_pair(k, first=False)
        return carry

    lax.fori_loop(1, NCH // 2, pair_body, 0)
    wait_write(rows0_v, wsem0)
    wait_write(rows1_v, wsem1)


def kernel(edge_attr, W0, W1, W2):
    ea = edge_attr.astype(jnp.int32)
    t = _build_table(W0, W1, W2)
    return _sc_lookup(ea[:, 0], ea[:, 1], ea[:, 2], t)
